# baseline (speedup 1.0000x reference)
import functools

import jax
import jax.numpy as jnp
from jax import lax
from jax.experimental import pallas as pl
from jax.experimental.pallas import tpu as pltpu
from jax.experimental.pallas import tpu_sc as plsc

N_NODES = 10000
N_EDGES = 160000
H_SIZE = 256
HALF = 128

NPAD = 10240
ROWS_PER_SUB = NPAD // 16
LAST_ROWS = N_NODES - 15 * ROWS_PER_SUB
EDGES_PER_SUB = N_EDGES // 16
EPAD = 240
K = 80
NCHUNK = (EDGES_PER_SUB + EPAD) // K


def _sc_body(h2_hbm, gidx_hbm, sidx_hbm, zeros_hbm, out_hbm,
             acc_sh, gidx_a, gidx_b, sidx_a, sidx_b, rows_a, rows_b,
             sem_a, sem_b):
    c = lax.axis_index("c")
    s = lax.axis_index("s")
    row0 = s * ROWS_PER_SUB

    pltpu.sync_copy(zeros_hbm, acc_sh.at[pl.ds(row0, ROWS_PER_SUB)])
    plsc.subcore_barrier()

    pltpu.sync_copy(gidx_hbm.at[c, s, 0], gidx_a)
    pltpu.sync_copy(sidx_hbm.at[s, 0], sidx_a)
    pltpu.async_copy(h2_hbm.at[gidx_a], rows_a, sem_a)

    def body(j, carry):
        i0 = 2 * j
        i1 = i0 + 1
        more = j < NCHUNK // 2 - 1
        pltpu.sync_copy(gidx_hbm.at[c, s, i1], gidx_b)
        pltpu.sync_copy(sidx_hbm.at[s, i1], sidx_b)
        pltpu.make_async_copy(h2_hbm.at[gidx_a], rows_a, sem_a).wait()
        pltpu.async_copy(h2_hbm.at[gidx_b], rows_b, sem_b)
        pltpu.sync_copy(rows_a, acc_sh.at[sidx_a], add=True)

        @pl.when(more)
        def _():
            pltpu.sync_copy(gidx_hbm.at[c, s, i0 + 2], gidx_a)
            pltpu.sync_copy(sidx_hbm.at[s, i0 + 2], sidx_a)

        pltpu.make_async_copy(h2_hbm.at[gidx_b], rows_b, sem_b).wait()

        @pl.when(more)
        def _():
            pltpu.async_copy(h2_hbm.at[gidx_a], rows_a, sem_a)

        pltpu.sync_copy(rows_b, acc_sh.at[sidx_b], add=True)
        return carry

    lax.fori_loop(0, NCHUNK // 2, body, 0)
    plsc.subcore_barrier()

    out0 = c * N_NODES + row0

    @pl.when(s < 15)
    def _():
        pltpu.sync_copy(acc_sh.at[pl.ds(row0, ROWS_PER_SUB)],
                        out_hbm.at[pl.ds(out0, ROWS_PER_SUB)])

    @pl.when(s == 15)
    def _():
        pltpu.sync_copy(acc_sh.at[pl.ds(row0, LAST_ROWS)],
                        out_hbm.at[pl.ds(out0, LAST_ROWS)])


_sc_segment_sum = functools.partial(
    pl.kernel,
    out_type=jax.ShapeDtypeStruct((2 * N_NODES, HALF), jnp.float32),
    mesh=plsc.VectorSubcoreMesh(core_axis_name="c", subcore_axis_name="s"),
    scratch_types=[
        pltpu.VMEM_SHARED((NPAD, HALF), jnp.float32),
        pltpu.VMEM((K,), jnp.int32),
        pltpu.VMEM((K,), jnp.int32),
        pltpu.VMEM((K,), jnp.int32),
        pltpu.VMEM((K,), jnp.int32),
        pltpu.VMEM((K, HALF), jnp.float32),
        pltpu.VMEM((K, HALF), jnp.float32),
        pltpu.SemaphoreType.DMA,
        pltpu.SemaphoreType.DMA,
    ],
)(_sc_body)


ROW_BLK = 1000


def _tc_body(x_ref, ht0_ref, ht1_ref, w_ref, u_ref, b_ref, out_ref):
    ht = jnp.concatenate([ht0_ref[...], ht1_ref[...]], axis=1)
    ruo = (jnp.dot(x_ref[...], w_ref[:, H_SIZE:],
                   preferred_element_type=jnp.float32)
           + jnp.dot(ht, u_ref[:, H_SIZE:],
                     preferred_element_type=jnp.float32)
           + b_ref[:, H_SIZE:])
    u = jax.nn.sigmoid(ruo[:, :H_SIZE])
    o = jnp.tanh(ruo[:, H_SIZE:])
    out_ref[...] = o * u + (1.0 - u) * ht


_tc_dense = pl.pallas_call(
    _tc_body,
    out_shape=jax.ShapeDtypeStruct((N_NODES, H_SIZE), jnp.float32),
    grid=(N_NODES // ROW_BLK,),
    in_specs=[
        pl.BlockSpec((ROW_BLK, H_SIZE), lambda i: (i, 0)),
        pl.BlockSpec((ROW_BLK, HALF), lambda i: (i, 0)),
        pl.BlockSpec((ROW_BLK, HALF), lambda i: (i + 10, 0)),
        pl.BlockSpec((H_SIZE, 3 * H_SIZE), lambda i: (0, 0)),
        pl.BlockSpec((H_SIZE, 3 * H_SIZE), lambda i: (0, 0)),
        pl.BlockSpec((1, 3 * H_SIZE), lambda i: (0, 0)),
    ],
    out_specs=pl.BlockSpec((ROW_BLK, H_SIZE), lambda i: (i, 0)),
)


def kernel(x, h, edge_index, W_ruo, U_ruo, b_ruo):
    src = edge_index[0].astype(jnp.int32)
    dst = edge_index[1].astype(jnp.int32)

    g0 = src * 2
    gidx = jnp.stack([g0, g0 + 1]).reshape(2, 16, EDGES_PER_SUB)
    gidx = jnp.pad(gidx, ((0, 0), (0, 0), (0, EPAD)))
    gidx = gidx.reshape(2, 16, NCHUNK, K)
    sidx = jnp.pad(dst.reshape(16, EDGES_PER_SUB), ((0, 0), (0, EPAD)),
                   constant_values=N_NODES)
    sidx = sidx.reshape(16, NCHUNK, K)

    h2 = h.reshape(2 * N_NODES, HALF)
    zeros = jnp.zeros((ROWS_PER_SUB, HALF), jnp.float32)

    ht_flat = _sc_segment_sum(h2, gidx, sidx, zeros)
    return _tc_dense(x, ht_flat, ht_flat, W_ruo, U_ruo, b_ruo)


# speedup vs baseline: 1.4163x; 1.4163x over previous
import functools

import jax
import jax.numpy as jnp
from jax import lax
from jax.experimental import pallas as pl
from jax.experimental.pallas import tpu as pltpu
from jax.experimental.pallas import tpu_sc as plsc

N_NODES = 10000
N_EDGES = 160000
H_SIZE = 256
HALF = 128

NPAD = 10240
ROWS_PER_SUB = NPAD // 16
LAST_ROWS = N_NODES - 15 * ROWS_PER_SUB
EDGES_PER_SUB = N_EDGES // 16
K = 80
NCHUNK = EDGES_PER_SUB // K


def _sc_body(h2_hbm, gidx_hbm, sidx_hbm, zeros_hbm, out_hbm,
             acc_sh, gidx_v, sidx_v, rows_v, sem):
    c = lax.axis_index("c")
    s = lax.axis_index("s")
    row0 = s * ROWS_PER_SUB

    pltpu.sync_copy(zeros_hbm, acc_sh.at[pl.ds(row0, ROWS_PER_SUB)])
    plsc.subcore_barrier()

    def body(i, carry):
        pltpu.sync_copy(gidx_hbm.at[c, s, i], gidx_v)
        pltpu.sync_copy(sidx_hbm.at[s, i], sidx_v)
        pltpu.async_copy(h2_hbm.at[gidx_v], rows_v, sem).wait()
        pltpu.sync_copy(rows_v, acc_sh.at[sidx_v], add=True)
        return carry

    lax.fori_loop(0, NCHUNK, body, 0)
    plsc.subcore_barrier()

    out0 = c * N_NODES + row0

    @pl.when(s < 15)
    def _():
        pltpu.sync_copy(acc_sh.at[pl.ds(row0, ROWS_PER_SUB)],
                        out_hbm.at[pl.ds(out0, ROWS_PER_SUB)])

    @pl.when(s == 15)
    def _():
        pltpu.sync_copy(acc_sh.at[pl.ds(row0, LAST_ROWS)],
                        out_hbm.at[pl.ds(out0, LAST_ROWS)])


_sc_segment_sum = functools.partial(
    pl.kernel,
    out_type=jax.ShapeDtypeStruct((2 * N_NODES, HALF), jnp.float32),
    mesh=plsc.VectorSubcoreMesh(core_axis_name="c", subcore_axis_name="s"),
    scratch_types=[
        pltpu.VMEM_SHARED((NPAD, HALF), jnp.float32),
        pltpu.VMEM((K,), jnp.int32),
        pltpu.VMEM((K,), jnp.int32),
        pltpu.VMEM((K, HALF), jnp.float32),
        pltpu.SemaphoreType.DMA,
    ],
)(_sc_body)


ROW_BLK = 1000


def _tc_body(x_ref, ht0_ref, ht1_ref, w_ref, u_ref, b_ref, out_ref):
    ht = jnp.concatenate([ht0_ref[...], ht1_ref[...]], axis=1)
    ruo = (jnp.dot(x_ref[...], w_ref[:, H_SIZE:],
                   preferred_element_type=jnp.float32)
           + jnp.dot(ht, u_ref[:, H_SIZE:],
                     preferred_element_type=jnp.float32)
           + b_ref[:, H_SIZE:])
    u = jax.nn.sigmoid(ruo[:, :H_SIZE])
    o = jnp.tanh(ruo[:, H_SIZE:])
    out_ref[...] = o * u + (1.0 - u) * ht


_tc_dense = pl.pallas_call(
    _tc_body,
    out_shape=jax.ShapeDtypeStruct((N_NODES, H_SIZE), jnp.float32),
    grid=(N_NODES // ROW_BLK,),
    in_specs=[
        pl.BlockSpec((ROW_BLK, H_SIZE), lambda i: (i, 0)),
        pl.BlockSpec((ROW_BLK, HALF), lambda i: (i, 0)),
        pl.BlockSpec((ROW_BLK, HALF), lambda i: (i + 10, 0)),
        pl.BlockSpec((H_SIZE, 3 * H_SIZE), lambda i: (0, 0)),
        pl.BlockSpec((H_SIZE, 3 * H_SIZE), lambda i: (0, 0)),
        pl.BlockSpec((1, 3 * H_SIZE), lambda i: (0, 0)),
    ],
    out_specs=pl.BlockSpec((ROW_BLK, H_SIZE), lambda i: (i, 0)),
)


def kernel(x, h, edge_index, W_ruo, U_ruo, b_ruo):
    src = edge_index[0].astype(jnp.int32)
    dst = edge_index[1].astype(jnp.int32)

    g0 = src * 2
    gidx = jnp.stack([g0, g0 + 1]).reshape(2, 16, NCHUNK, K)
    sidx = dst.reshape(16, NCHUNK, K)

    h2 = h.reshape(2 * N_NODES, HALF)
    zeros = jnp.zeros((ROWS_PER_SUB, HALF), jnp.float32)

    ht_flat = _sc_segment_sum(h2, gidx, sidx, zeros)
    return _tc_dense(x, ht_flat, ht_flat, W_ruo, U_ruo, b_ruo)
